# baseline (speedup 1.0000x reference)
import functools

import jax
import jax.numpy as jnp
from jax import lax
from jax.experimental import pallas as pl
from jax.experimental.pallas import tpu as pltpu
from jax.experimental.pallas import tpu_sc as plsc

_N = 10000
_K = 32
_C = 128
_NSC = 2
_NT = 16
_CPT = _C // _NT
_WPT = _CPT // 2
_NPAD = 10240
_NPC = _NPAD // _NSC
_CN = 256
_NCHK = _NPC // _CN
_G = _CN // 16


def _sc_max_rel(x_shard, idx_prep):
    mesh = plsc.VectorSubcoreMesh(core_axis_name="c", subcore_axis_name="s")

    @functools.partial(
        pl.kernel,
        mesh=mesh,
        out_type=jax.ShapeDtypeStruct((_NT, _WPT, _NSC, _NCHK, _CN),
                                      jnp.int32),
        compiler_params=pltpu.CompilerParams(needs_layout_passes=False),
        scratch_types=[
            pltpu.VMEM((_NPAD * _WPT,), jnp.int32),
            pltpu.VMEM((2, _K, _CN), jnp.int32),
            pltpu.VMEM((2, _K, _CN), jnp.int32),
            pltpu.VMEM((_WPT, _CN), jnp.int32),
            pltpu.VMEM((_WPT, _CN), jnp.int32),
            pltpu.SemaphoreType.DMA,
            pltpu.SemaphoreType.DMA,
            pltpu.SemaphoreType.DMA,
            pltpu.SemaphoreType.DMA,
        ],
    )
    def sc_kernel(x_hbm, idx_hbm, out_hbm, table, ib0, ib1, ob0, ob1,
                  si0, si1, so0, so1):
        c = lax.axis_index("c")
        t = lax.axis_index("s")
        ib = (ib0, ib1)
        ob = (ob0, ob1)
        si = (si0, si1)
        so = (so0, so1)
        w_vecs = [jnp.full((16,), w * _NPAD, jnp.int32) for w in range(_WPT)]

        pltpu.sync_copy(x_hbm.at[t], table)

        def fire_idx(ci, j):
            pltpu.async_copy(idx_hbm.at[c, ci], ib[j], si[j])

        def wait_idx(j):
            pltpu.make_async_copy(idx_hbm.at[c, 0], ib[j], si[j]).wait()

        def fire_out(ci, j):
            pltpu.async_copy(ob[j], out_hbm.at[t, :, c, ci], so[j])

        def wait_out(j):
            pltpu.make_async_copy(ob[j], out_hbm.at[t, :, c, 0], so[j]).wait()

        def compute(j):
            ib_ = ib[j]
            ob_ = ob[j]

            def group_body(g, carry):
                n0 = g * 16
                accs = None
                for k in range(_K):
                    isv = ib_[0, k, pl.ds(n0, 16)]
                    idv = ib_[1, k, pl.ds(n0, 16)]
                    new = []
                    for w in range(_WPT):
                        avs = isv if w == 0 else isv + w_vecs[w]
                        avd = idv if w == 0 else idv + w_vecs[w]
                        s = plsc.bitcast(
                            plsc.load_gather(table, [avs]), jnp.bfloat16)
                        d = plsc.bitcast(
                            plsc.load_gather(table, [avd]), jnp.bfloat16)
                        diff = s - d
                        if accs is None:
                            new.append(diff)
                        else:
                            new.append(jnp.maximum(accs[w], diff))
                    accs = new
                for w in range(_WPT):
                    ob_[w, pl.ds(n0, 16)] = plsc.bitcast(accs[w], jnp.int32)
                return carry

            lax.fori_loop(0, _G, group_body, 0)

        fire_idx(0, 0)

        def pair_body(p, carry):
            for j in range(2):
                ci = 2 * p + j
                wait_idx(j)

                @pl.when(ci + 1 < _NCHK)
                def _(ci=ci, j=j):
                    fire_idx(ci + 1, 1 - j)

                @pl.when(p >= 1)
                def _(j=j):
                    wait_out(j)

                compute(j)
                fire_out(ci, j)
            return carry

        lax.fori_loop(0, _NCHK // 2, pair_body, 0)
        wait_out(0)
        wait_out(1)

    return sc_kernel(x_shard, idx_prep)


_BN = 1024
_NBLK = _NPAD // _BN


def _tc_fuse(x2p, xq, a, b_lo, b_hi, gamma, beta):
    def body(x2_ref, xq_ref, a_ref, bl_ref, bh_ref, g_ref, bt_ref, o_ref,
             y_scr, s_scr, q_scr):
        ph = pl.program_id(0)
        blk = pl.program_id(1)

        @pl.when(ph == 0)
        def _():
            xw = xq_ref[...]
            xlo = lax.bitcast_convert_type(xw << 16, jnp.float32)
            xhi = lax.bitcast_convert_type(xw & jnp.int32(-65536),
                                           jnp.float32)
            nn = (((1,), (0,)), ((), ()))
            y = lax.dot_general(a_ref[...], x2_ref[...], nn,
                                precision=lax.Precision.HIGHEST,
                                preferred_element_type=jnp.float32)
            y = y + lax.dot_general(bl_ref[...], xlo, nn,
                                    precision=lax.Precision.HIGHEST,
                                    preferred_element_type=jnp.float32)
            y = y + lax.dot_general(bh_ref[...], xhi, nn,
                                    precision=lax.Precision.HIGHEST,
                                    preferred_element_type=jnp.float32)
            y_scr[:, pl.ds(blk * _BN, _BN)] = y

            @pl.when(blk == 0)
            def _():
                s_scr[...] = jnp.zeros_like(s_scr)
                q_scr[...] = jnp.zeros_like(q_scr)

            s_scr[...] += jnp.sum(y, axis=1, keepdims=True)
            q_scr[...] += jnp.sum(y * y, axis=1, keepdims=True)

        @pl.when(ph == 1)
        def _():
            y = y_scr[:, pl.ds(blk * _BN, _BN)]
            mean = s_scr[...] * (1.0 / _N)
            var = q_scr[...] * (1.0 / _N) - mean * mean
            scale = g_ref[...] * lax.rsqrt(var + 1e-5)
            o_ref[...] = jnp.maximum((y - mean) * scale + bt_ref[...], 0.0)

    const = lambda ph, blk: (0, 0)
    phase0 = lambda ph, blk: (0, blk * (1 - ph))
    return pl.pallas_call(
        body,
        grid=(2, _NBLK),
        in_specs=[
            pl.BlockSpec((_C, _BN), phase0),
            pl.BlockSpec((_C // 2, _BN), phase0),
            pl.BlockSpec((_C, _C), const),
            pl.BlockSpec((_C, _C // 2), const),
            pl.BlockSpec((_C, _C // 2), const),
            pl.BlockSpec((_C, 1), const),
            pl.BlockSpec((_C, 1), const),
        ],
        out_specs=pl.BlockSpec((_C, _BN), lambda ph, blk: (0, blk * ph)),
        scratch_shapes=[
            pltpu.VMEM((_C, _NPAD), jnp.float32),
            pltpu.VMEM((_C, 1), jnp.float32),
            pltpu.VMEM((_C, 1), jnp.float32),
        ],
        out_shape=jax.ShapeDtypeStruct((_C, _NPAD), jnp.float32),
    )(x2p, xq, a, b_lo, b_hi, gamma, beta)


def kernel(x, edge_index, conv_w, conv_b, bn_gamma, bn_beta):
    del conv_b
    x2 = x[0, :, :, 0]
    pairs = x2.astype(jnp.bfloat16).reshape(_C // 2, 2, _N)
    lo = lax.bitcast_convert_type(pairs[:, 0], jnp.uint16).astype(jnp.uint32)
    hi = lax.bitcast_convert_type(pairs[:, 1], jnp.uint16).astype(jnp.uint32)
    words = lax.bitcast_convert_type(lo | (hi << 16), jnp.int32)
    x_shard = jnp.pad(words, ((0, 0), (0, _NPAD - _N))).reshape(
        _NT, _WPT * _NPAD)
    ei_p = jnp.pad(edge_index, ((0, 0), (0, _NPAD - _N), (0, 0)))
    idx_prep = (ei_p.transpose(0, 2, 1)
                .reshape(2, _K, _NSC, _NCHK, _CN)
                .transpose(2, 3, 0, 1, 4))
    outp = _sc_max_rel(x_shard, idx_prep)
    xq = outp.reshape(_C // 2, _NPAD)
    w = conv_w.reshape(4, 32, 32, 2)
    eye = jnp.eye(4, dtype=conv_w.dtype)
    a = jnp.einsum('goc,gh->gohc', w[..., 0], eye).reshape(_C, _C)
    b = jnp.einsum('goc,gh->gohc', w[..., 1], eye).reshape(_C, _C)
    b_pairs = b.reshape(_C, _C // 2, 2)
    b_lo = b_pairs[:, :, 0]
    b_hi = b_pairs[:, :, 1]
    x2p = jnp.pad(x2, ((0, 0), (0, _NPAD - _N)))
    y = _tc_fuse(x2p, xq, a, b_lo, b_hi,
                 bn_gamma.reshape(_C, 1), bn_beta.reshape(_C, 1))
    return y[None, :, :10000, None]


# speedup vs baseline: 1.0470x; 1.0470x over previous
import functools

import jax
import jax.numpy as jnp
from jax import lax
from jax.experimental import pallas as pl
from jax.experimental.pallas import tpu as pltpu
from jax.experimental.pallas import tpu_sc as plsc

_N = 10000
_K = 32
_C = 128
_NSC = 2
_NT = 16
_CPT = _C // _NT
_WPT = _CPT // 2
_NPAD = 10240
_NPC = _NPAD // _NSC
_CN = 128
_NCHK = _NPC // _CN
_G = _CN // 16


def _sc_max_rel(x_shard, idx_prep):
    mesh = plsc.VectorSubcoreMesh(core_axis_name="c", subcore_axis_name="s")

    @functools.partial(
        pl.kernel,
        mesh=mesh,
        out_type=jax.ShapeDtypeStruct((_NT, _WPT, _NSC, _NCHK, _CN),
                                      jnp.int32),
        compiler_params=pltpu.CompilerParams(needs_layout_passes=False),
        scratch_types=[
            pltpu.VMEM((_NPAD * _WPT,), jnp.int32),
            pltpu.VMEM((2, _K, _CN), jnp.int32),
            pltpu.VMEM((2, _K, _CN), jnp.int32),
            pltpu.VMEM((_WPT, _CN), jnp.int32),
            pltpu.VMEM((_WPT, _CN), jnp.int32),
            pltpu.SemaphoreType.DMA,
            pltpu.SemaphoreType.DMA,
            pltpu.SemaphoreType.DMA,
            pltpu.SemaphoreType.DMA,
        ],
    )
    def sc_kernel(x_hbm, idx_hbm, out_hbm, table, ib0, ib1, ob0, ob1,
                  si0, si1, so0, so1):
        c = lax.axis_index("c")
        t = lax.axis_index("s")
        ib = (ib0, ib1)
        ob = (ob0, ob1)
        si = (si0, si1)
        so = (so0, so1)
        w_vecs = [jnp.full((16,), w * _NPAD, jnp.int32) for w in range(_WPT)]

        pltpu.sync_copy(x_hbm.at[t], table)

        def fire_idx(ci, j):
            pltpu.async_copy(idx_hbm.at[c, ci], ib[j], si[j])

        def wait_idx(j):
            pltpu.make_async_copy(idx_hbm.at[c, 0], ib[j], si[j]).wait()

        def fire_out(ci, j):
            pltpu.async_copy(ob[j], out_hbm.at[t, :, c, ci], so[j])

        def wait_out(j):
            pltpu.make_async_copy(ob[j], out_hbm.at[t, :, c, 0], so[j]).wait()

        def compute(j):
            ib_ = ib[j]
            ob_ = ob[j]

            def group_body(g, carry):
                n0 = g * 16
                accs = None
                for k in range(_K):
                    isv = ib_[0, k, pl.ds(n0, 16)]
                    idv = ib_[1, k, pl.ds(n0, 16)]
                    new = []
                    for w in range(_WPT):
                        avs = isv if w == 0 else isv + w_vecs[w]
                        avd = idv if w == 0 else idv + w_vecs[w]
                        s = plsc.bitcast(
                            plsc.load_gather(table, [avs]), jnp.bfloat16)
                        d = plsc.bitcast(
                            plsc.load_gather(table, [avd]), jnp.bfloat16)
                        diff = s - d
                        if accs is None:
                            new.append(diff)
                        else:
                            new.append(jnp.maximum(accs[w], diff))
                    accs = new
                for w in range(_WPT):
                    ob_[w, pl.ds(n0, 16)] = plsc.bitcast(accs[w], jnp.int32)
                return carry

            lax.fori_loop(0, _G, group_body, 0)

        fire_idx(0, 0)

        def pair_body(p, carry):
            for j in range(2):
                ci = 2 * p + j
                wait_idx(j)

                @pl.when(ci + 1 < _NCHK)
                def _(ci=ci, j=j):
                    fire_idx(ci + 1, 1 - j)

                @pl.when(p >= 1)
                def _(j=j):
                    wait_out(j)

                compute(j)
                fire_out(ci, j)
            return carry

        lax.fori_loop(0, _NCHK // 2, pair_body, 0)
        wait_out(0)
        wait_out(1)

    return sc_kernel(x_shard, idx_prep)


_BN = 1024
_NBLK = _NPAD // _BN


def _tc_fuse(x2p, xq, a, b_lo, b_hi, gamma, beta):
    def body(x2_ref, xq_ref, a_ref, bl_ref, bh_ref, g_ref, bt_ref, o_ref,
             y_scr, s_scr, q_scr):
        ph = pl.program_id(0)
        blk = pl.program_id(1)

        @pl.when(ph == 0)
        def _():
            xw = xq_ref[...]
            xlo = lax.bitcast_convert_type(xw << 16, jnp.float32)
            xhi = lax.bitcast_convert_type(xw & jnp.int32(-65536),
                                           jnp.float32)
            nn = (((1,), (0,)), ((), ()))
            y = lax.dot_general(a_ref[...], x2_ref[...], nn,
                                precision=lax.Precision.HIGHEST,
                                preferred_element_type=jnp.float32)
            y = y + lax.dot_general(bl_ref[...], xlo, nn,
                                    precision=lax.Precision.HIGHEST,
                                    preferred_element_type=jnp.float32)
            y = y + lax.dot_general(bh_ref[...], xhi, nn,
                                    precision=lax.Precision.HIGHEST,
                                    preferred_element_type=jnp.float32)
            y_scr[:, pl.ds(blk * _BN, _BN)] = y

            @pl.when(blk == 0)
            def _():
                s_scr[...] = jnp.zeros_like(s_scr)
                q_scr[...] = jnp.zeros_like(q_scr)

            s_scr[...] += jnp.sum(y, axis=1, keepdims=True)
            q_scr[...] += jnp.sum(y * y, axis=1, keepdims=True)

        @pl.when(ph == 1)
        def _():
            y = y_scr[:, pl.ds(blk * _BN, _BN)]
            mean = s_scr[...] * (1.0 / _N)
            var = q_scr[...] * (1.0 / _N) - mean * mean
            scale = g_ref[...] * lax.rsqrt(var + 1e-5)
            o_ref[...] = jnp.maximum((y - mean) * scale + bt_ref[...], 0.0)

    const = lambda ph, blk: (0, 0)
    phase0 = lambda ph, blk: (0, blk * (1 - ph))
    return pl.pallas_call(
        body,
        grid=(2, _NBLK),
        in_specs=[
            pl.BlockSpec((_C, _BN), phase0),
            pl.BlockSpec((_C // 2, _BN), phase0),
            pl.BlockSpec((_C, _C), const),
            pl.BlockSpec((_C, _C // 2), const),
            pl.BlockSpec((_C, _C // 2), const),
            pl.BlockSpec((_C, 1), const),
            pl.BlockSpec((_C, 1), const),
        ],
        out_specs=pl.BlockSpec((_C, _BN), lambda ph, blk: (0, blk * ph)),
        scratch_shapes=[
            pltpu.VMEM((_C, _NPAD), jnp.float32),
            pltpu.VMEM((_C, 1), jnp.float32),
            pltpu.VMEM((_C, 1), jnp.float32),
        ],
        out_shape=jax.ShapeDtypeStruct((_C, _NPAD), jnp.float32),
    )(x2p, xq, a, b_lo, b_hi, gamma, beta)


def kernel(x, edge_index, conv_w, conv_b, bn_gamma, bn_beta):
    del conv_b
    x2 = x[0, :, :, 0]
    pairs = x2.astype(jnp.bfloat16).reshape(_C // 2, 2, _N)
    lo = lax.bitcast_convert_type(pairs[:, 0], jnp.uint16).astype(jnp.uint32)
    hi = lax.bitcast_convert_type(pairs[:, 1], jnp.uint16).astype(jnp.uint32)
    words = lax.bitcast_convert_type(lo | (hi << 16), jnp.int32)
    x_shard = jnp.pad(words, ((0, 0), (0, _NPAD - _N))).reshape(
        _NT, _WPT * _NPAD)
    ei_p = jnp.pad(edge_index, ((0, 0), (0, _NPAD - _N), (0, 0)))
    idx_prep = (ei_p.transpose(0, 2, 1)
                .reshape(2, _K, _NSC, _NCHK, _CN)
                .transpose(2, 3, 0, 1, 4))
    outp = _sc_max_rel(x_shard, idx_prep)
    xq = outp.reshape(_C // 2, _NPAD)
    w = conv_w.reshape(4, 32, 32, 2)
    eye = jnp.eye(4, dtype=conv_w.dtype)
    a = jnp.einsum('goc,gh->gohc', w[..., 0], eye).reshape(_C, _C)
    b = jnp.einsum('goc,gh->gohc', w[..., 1], eye).reshape(_C, _C)
    b_pairs = b.reshape(_C, _C // 2, 2)
    b_lo = b_pairs[:, :, 0]
    b_hi = b_pairs[:, :, 1]
    x2p = jnp.pad(x2, ((0, 0), (0, _NPAD - _N)))
    y = _tc_fuse(x2p, xq, a, b_lo, b_hi,
                 bn_gamma.reshape(_C, 1), bn_beta.reshape(_C, 1))
    return y[None, :, :10000, None]
